# baseline (speedup 1.0000x reference)
import jax
import jax.numpy as jnp
from jax.experimental import pallas as pl
from jax.experimental.pallas import tpu as pltpu

_STATIC_LEN = 8192
_HIDDEN = 1024
_CHUNKS = (128, 128, 256, 512, 1024, 1024, 1024, 1024, 1024, 1024, 1024)
assert sum(_CHUNKS) == _STATIC_LEN
_N = len(_CHUNKS)
_OFFS = tuple(sum(_CHUNKS[:i]) for i in range(_N))


def _copy_body(start_ref, src_ref, out_ref, *rest):
    bufs = rest[:_N]
    lsems = rest[_N:2 * _N]
    ssems = rest[2 * _N:3 * _N]
    start = pl.multiple_of(start_ref[0], 8)

    loads = [
        pltpu.async_copy(
            src_ref.at[pl.ds(start + _OFFS[g], _CHUNKS[g]), :],
            bufs[g], lsems[g])
        for g in range(_N)
    ]
    stores = []
    for g in range(_N):
        loads[g].wait()
        stores.append(pltpu.async_copy(
            bufs[g], out_ref.at[pl.ds(_OFFS[g], _CHUNKS[g]), :], ssems[g]))
    for s in stores:
        s.wait()


@jax.jit
def kernel(freqs, seq_len):
    src = freqs.reshape(_STATIC_LEN * 8, _HIDDEN)
    start = (jnp.asarray(seq_len, jnp.int32) - _STATIC_LEN).reshape(1)
    out = pl.pallas_call(
        _copy_body,
        out_shape=jax.ShapeDtypeStruct((_STATIC_LEN, _HIDDEN), jnp.float32),
        in_specs=[
            pl.BlockSpec(memory_space=pltpu.SMEM),
            pl.BlockSpec(memory_space=pl.ANY),
        ],
        out_specs=pl.BlockSpec(memory_space=pl.ANY),
        scratch_shapes=(
            [pltpu.VMEM((c, _HIDDEN), jnp.float32) for c in _CHUNKS]
            + [pltpu.SemaphoreType.DMA] * (2 * _N)
        ),
    )(start, src)
    return out.reshape(1, _STATIC_LEN, _HIDDEN)
